# baseline (speedup 1.0000x reference)
import math

import jax
import jax.numpy as jnp
from jax.experimental import pallas as pl
from jax.experimental.pallas import tpu as pltpu

_BF16 = jnp.bfloat16
_SQRT2 = math.sqrt(2.0)

_CH = 512
_H = 8
_HD = 64
_NQ, _NA, _NC = 512, 1024, 768
_NTOT = _NQ + _NA + _NC
_ROWS = {"question": (0, 512), "answer": (512, 1536), "concept": (1536, 2304)}
_NEG = -1e30
_TYPES = ("question", "answer", "concept")
_ETS = (("has", "question", "answer"),
        ("rev_has", "answer", "question"),
        ("mentions", "question", "concept"),
        ("rev_mentions", "concept", "question"))


def _erf(x):
    a1, a2, a3, a4, a5 = 0.254829592, -0.284496736, 1.421413741, -1.453152027, 1.061405429
    p = 0.3275911
    sgn = jnp.where(x >= 0.0, 1.0, -1.0)
    ax = jnp.abs(x)
    t = 1.0 / (1.0 + p * ax)
    poly = ((((a5 * t + a4) * t + a3) * t + a2) * t + a1) * t
    return sgn * (1.0 - poly * jnp.exp(-ax * ax))


def _gelu_exact(x):
    return 0.5 * x * (1.0 + _erf(x / _SQRT2))


def _dot(a, b):
    return jnp.dot(a, b, preferred_element_type=jnp.float32)


def _dot_nt(a, b):
    return jax.lax.dot_general(a, b, (((1,), (1,)), ((), ())),
                               preferred_element_type=jnp.float32)


def _attend(kbase, vbase, srcs, nd, qbuf, lc, alin_w, alin_b, alpha, hd, write,
            sbuf, sb16, abuf, q2b, araw, bdbuf):
    for si, (s0, s1, c0, rk, rv) in enumerate(srcs):
        ns = s1 - s0
        tb = sbuf.at[:, 0:ns]
        wb = sb16.at[:, 0:ns]
        for h in range(_H):
            bd = slice(h * _HD, (h + 1) * _HD)
            bdbuf[bd, bd] = rk[h]
        q2b[...] = _dot_nt(qbuf[...], bdbuf[...]).astype(_BF16)
        for h in range(_H):
            sl = slice(h * _HD, (h + 1) * _HD)
            tb[...] = _dot_nt(q2b[:, sl], kbase[s0:s1, sl]) + lc[:, c0:c0 + ns]
            rm = jnp.max(tb[...], axis=-1, keepdims=True)
            ok = rm > -1e29
            wb[...] = jnp.exp(tb[...] - rm).astype(_BF16)
            denom = jnp.sum(wb[...].astype(jnp.float32), axis=-1, keepdims=True)
            inv = jnp.where(ok, 1.0 / denom, 0.0)
            o = _dot(wb[...], vbase[s0:s1, sl]) * inv
            araw[:, sl] = o.astype(_BF16)
        for h in range(_H):
            bd = slice(h * _HD, (h + 1) * _HD)
            bdbuf[bd, bd] = rv[h]
        oet = _dot(araw[...], bdbuf[...])
        if si == 0:
            abuf[...] = oet
        else:
            abuf[...] = abuf[...] + oet
    att = abuf[...]
    g = _gelu_exact(att).astype(_BF16)
    y = _dot(g, alin_w[...]) + alin_b[...]
    a = alpha[...]
    write(a * y + (1.0 - a) * hd.astype(jnp.float32))


def _build_lc(e_ref, nd, ns, out_ref, col0):
    ne = e_ref.shape[1]
    dt = jnp.float8_e4m3fn

    def f(a_ref, b_ref):
        a_ref[...] = (jax.lax.broadcasted_iota(jnp.int32, (nd, ne), 0)
                      == e_ref[1:2, :]).astype(dt)
        b_ref[...] = (jax.lax.broadcasted_iota(jnp.int32, (ns, ne), 0)
                      == e_ref[0:1, :]).astype(dt)
        cnt = _dot_nt(a_ref[...], b_ref[...])
        out_ref[:, col0:col0 + ns] = jnp.where(cnt > 0.0, jnp.log(cnt), _NEG)

    pl.run_scoped(f, pltpu.VMEM((nd, ne), dt), pltpu.VMEM((ns, ne), dt))


def _body(xq, xa, xc, w_lin, w_all, rel, rows,
          e_has, e_rev_has, e_mentions, e_rev_mentions,
          out_q, out_a, out_c, hb0, hb1, kbase, vbase, bdbuf,
          lc_q, lc_a, lc_c):
    xs = {"question": xq, "answer": xa, "concept": xc}
    bdbuf[...] = jnp.zeros((_CH, _CH), _BF16)
    lin_w = {"question": w_lin.at[0:256], "answer": w_lin.at[256:384],
             "concept": w_lin.at[384:512]}

    edges = {"has": e_has, "rev_has": e_rev_has, "mentions": e_mentions,
             "rev_mentions": e_rev_mentions}
    lc_of = {"question": lc_q, "answer": lc_a, "concept": lc_c}
    col0 = {"has": 0, "rev_has": 0, "mentions": 0, "rev_mentions": _NA}
    _build_lc(edges["rev_has"], _NQ, _NA, lc_q, 0)
    _build_lc(edges["rev_mentions"], _NQ, _NC, lc_q, _NA)
    _build_lc(edges["has"], _NA, _NQ, lc_a, 0)
    _build_lc(edges["mentions"], _NC, _NQ, lc_c, 0)

    for i, t in enumerate(_TYPES):
        r0, r1 = _ROWS[t]
        y = _dot(xs[t][...], lin_w[t][...]) + rows[i:i + 1]
        y = jnp.maximum(y, 0.0)
        n = r1 - r0
        mean = jnp.sum(y, axis=0, keepdims=True) * (1.0 / n)
        yc = y - mean
        var = jnp.sum(yc * yc, axis=0, keepdims=True) * (1.0 / n)
        y = yc * jax.lax.rsqrt(var + 1e-5) * rows[3 + i:4 + i] + rows[6 + i:7 + i]
        hb0[r0:r1] = y.astype(_BF16)

    for L, (hb_in, wr) in enumerate(((hb0, None), (hb1, None))):
        wb = 12 * L
        rb = 9 + 15 * L
        for i, t in enumerate(_TYPES):
            r0, r1 = _ROWS[t]
            h = hb_in[r0:r1]
            kbase[r0:r1] = (_dot(h, w_all[wb + 3 + i]) + rows[rb + 3 + i:rb + 4 + i]).astype(_BF16)
            vbase[r0:r1] = (_dot(h, w_all[wb + 6 + i]) + rows[rb + 6 + i:rb + 7 + i]).astype(_BF16)
        for i, t in enumerate(_TYPES):
            d0, d1 = _ROWS[t]
            nd = d1 - d0
            hd = hb_in[d0:d1]
            srcs = []
            for j, (et, s, d) in enumerate(_ETS):
                if d != t:
                    continue
                srcs.append((_ROWS[s][0], _ROWS[s][1], col0[et],
                             rel.at[8 * L + 2 * j], rel.at[8 * L + 2 * j + 1]))
            ns_max = max(s1 - s0 for s0, s1, _, _, _ in srcs)
            if L == 0:
                def write(v, _r0=d0, _r1=d1):
                    hb1[_r0:_r1] = v.astype(_BF16)
            else:
                out = {"question": out_q, "answer": out_a, "concept": out_c}[t]

                def write(v, _o=out):
                    _o[...] = v

            def scoped(sbuf, sb16, qbuf, abuf, q2b, araw, _i=i, _t=t, _hd=hd,
                       _srcs=srcs, _nd=nd, _write=write):
                qbuf[...] = (_dot(_hd, w_all[wb + _i])
                             + rows[rb + _i:rb + 1 + _i]).astype(_BF16)
                _attend(kbase, vbase, _srcs, _nd, qbuf, lc_of[_t],
                        w_all.at[wb + 9 + _i],
                        rows.at[rb + 9 + _i:rb + 10 + _i],
                        rows.at[rb + 12 + _i:rb + 13 + _i], _hd, _write,
                        sbuf, sb16, abuf, q2b, araw, bdbuf)

            pl.run_scoped(scoped,
                          pltpu.VMEM((nd, ns_max), jnp.float32),
                          pltpu.VMEM((nd, ns_max), _BF16),
                          pltpu.VMEM((nd, _CH), _BF16),
                          pltpu.VMEM((nd, _CH), jnp.float32),
                          pltpu.VMEM((nd, _CH), _BF16),
                          pltpu.VMEM((nd, _CH), _BF16))


def kernel(lin_w_question, lin_b_question, bn_gamma_question, bn_beta_question, lin_w_answer, lin_b_answer, bn_gamma_answer, bn_beta_answer, lin_w_concept, lin_b_concept, bn_gamma_concept, bn_beta_concept, c0_k_w_question, c0_k_b_question, c0_q_w_question, c0_q_b_question, c0_v_w_question, c0_v_b_question, c0_alin_w_question, c0_alin_b_question, c0_skip_question, c0_k_w_answer, c0_k_b_answer, c0_q_w_answer, c0_q_b_answer, c0_v_w_answer, c0_v_b_answer, c0_alin_w_answer, c0_alin_b_answer, c0_skip_answer, c0_k_w_concept, c0_k_b_concept, c0_q_w_concept, c0_q_b_concept, c0_v_w_concept, c0_v_b_concept, c0_alin_w_concept, c0_alin_b_concept, c0_skip_concept, c0_arel_question_has_answer, c0_mrel_question_has_answer, c0_prel_question_has_answer, c0_arel_answer_rev_has_question, c0_mrel_answer_rev_has_question, c0_prel_answer_rev_has_question, c0_arel_question_mentions_concept, c0_mrel_question_mentions_concept, c0_prel_question_mentions_concept, c0_arel_concept_rev_mentions_question, c0_mrel_concept_rev_mentions_question, c0_prel_concept_rev_mentions_question, c1_k_w_question, c1_k_b_question, c1_q_w_question, c1_q_b_question, c1_v_w_question, c1_v_b_question, c1_alin_w_question, c1_alin_b_question, c1_skip_question, c1_k_w_answer, c1_k_b_answer, c1_q_w_answer, c1_q_b_answer, c1_v_w_answer, c1_v_b_answer, c1_alin_w_answer, c1_alin_b_answer, c1_skip_answer, c1_k_w_concept, c1_k_b_concept, c1_q_w_concept, c1_q_b_concept, c1_v_w_concept, c1_v_b_concept, c1_alin_w_concept, c1_alin_b_concept, c1_skip_concept, c1_arel_question_has_answer, c1_mrel_question_has_answer, c1_prel_question_has_answer, c1_arel_answer_rev_has_question, c1_mrel_answer_rev_has_question, c1_prel_answer_rev_has_question, c1_arel_question_mentions_concept, c1_mrel_question_mentions_concept, c1_prel_question_mentions_concept, c1_arel_concept_rev_mentions_question, c1_mrel_concept_rev_mentions_question, c1_prel_concept_rev_mentions_question, x_question, x_answer, x_concept, edge_question_has_answer, edge_answer_rev_has_question, edge_question_mentions_concept, edge_concept_rev_mentions_question):
    c0 = {
        "k_w": (c0_k_w_question, c0_k_w_answer, c0_k_w_concept),
        "k_b": (c0_k_b_question, c0_k_b_answer, c0_k_b_concept),
        "q_w": (c0_q_w_question, c0_q_w_answer, c0_q_w_concept),
        "q_b": (c0_q_b_question, c0_q_b_answer, c0_q_b_concept),
        "v_w": (c0_v_w_question, c0_v_w_answer, c0_v_w_concept),
        "v_b": (c0_v_b_question, c0_v_b_answer, c0_v_b_concept),
        "alin_w": (c0_alin_w_question, c0_alin_w_answer, c0_alin_w_concept),
        "alin_b": (c0_alin_b_question, c0_alin_b_answer, c0_alin_b_concept),
        "skip": (c0_skip_question, c0_skip_answer, c0_skip_concept),
        "arel": (c0_arel_question_has_answer, c0_arel_answer_rev_has_question,
                 c0_arel_question_mentions_concept, c0_arel_concept_rev_mentions_question),
        "mrel": (c0_mrel_question_has_answer, c0_mrel_answer_rev_has_question,
                 c0_mrel_question_mentions_concept, c0_mrel_concept_rev_mentions_question),
        "prel": (c0_prel_question_has_answer, c0_prel_answer_rev_has_question,
                 c0_prel_question_mentions_concept, c0_prel_concept_rev_mentions_question),
    }
    c1 = {
        "k_w": (c1_k_w_question, c1_k_w_answer, c1_k_w_concept),
        "k_b": (c1_k_b_question, c1_k_b_answer, c1_k_b_concept),
        "q_w": (c1_q_w_question, c1_q_w_answer, c1_q_w_concept),
        "q_b": (c1_q_b_question, c1_q_b_answer, c1_q_b_concept),
        "v_w": (c1_v_w_question, c1_v_w_answer, c1_v_w_concept),
        "v_b": (c1_v_b_question, c1_v_b_answer, c1_v_b_concept),
        "alin_w": (c1_alin_w_question, c1_alin_w_answer, c1_alin_w_concept),
        "alin_b": (c1_alin_b_question, c1_alin_b_answer, c1_alin_b_concept),
        "skip": (c1_skip_question, c1_skip_answer, c1_skip_concept),
        "arel": (c1_arel_question_has_answer, c1_arel_answer_rev_has_question,
                 c1_arel_question_mentions_concept, c1_arel_concept_rev_mentions_question),
        "mrel": (c1_mrel_question_has_answer, c1_mrel_answer_rev_has_question,
                 c1_mrel_question_mentions_concept, c1_mrel_concept_rev_mentions_question),
        "prel": (c1_prel_question_has_answer, c1_prel_answer_rev_has_question,
                 c1_prel_question_mentions_concept, c1_prel_concept_rev_mentions_question),
    }

    w_all = jnp.stack(
        [w for cl in (c0, c1)
         for grp in ("q_w", "k_w", "v_w", "alin_w") for w in cl[grp]]
    ).astype(_BF16)
    w_lin = jnp.concatenate([lin_w_question, lin_w_answer,
                             lin_w_concept], axis=0).astype(_BF16)

    rel = jnp.stack(
        [r for cl in (c0, c1) for j in range(4)
         for r in (cl["arel"][j] * (cl["prel"][j] / math.sqrt(_HD))[:, None, None],
                   cl["mrel"][j])]).astype(_BF16)

    alphas = jax.nn.sigmoid(jnp.stack(list(c0["skip"]) + list(c1["skip"])))
    alpha_rows = jnp.broadcast_to(alphas[:, None], (6, _CH))
    rows = jnp.concatenate(
        [lin_b_question, lin_b_answer, lin_b_concept,
         bn_gamma_question, bn_gamma_answer, bn_gamma_concept,
         bn_beta_question, bn_beta_answer, bn_beta_concept]
        + [b for b in c0["q_b"] + c0["k_b"] + c0["v_b"] + c0["alin_b"]]
        + [alpha_rows[0:3]]
        + [b for b in c1["q_b"] + c1["k_b"] + c1["v_b"] + c1["alin_b"]]
        + [alpha_rows[3:6]], axis=0)

    ins = [x_question.astype(_BF16), x_answer.astype(_BF16),
           x_concept.astype(_BF16), w_lin, w_all, rel, rows,
           edge_question_has_answer, edge_answer_rev_has_question,
           edge_question_mentions_concept, edge_concept_rev_mentions_question]

    out = pl.pallas_call(
        _body,
        out_shape=(jax.ShapeDtypeStruct((_NQ, _CH), jnp.float32),
                   jax.ShapeDtypeStruct((_NA, _CH), jnp.float32),
                   jax.ShapeDtypeStruct((_NC, _CH), jnp.float32)),
        in_specs=[pl.BlockSpec(memory_space=pltpu.MemorySpace.VMEM)] * 11,
        out_specs=(pl.BlockSpec(memory_space=pltpu.MemorySpace.VMEM),) * 3,
        scratch_shapes=[pltpu.VMEM((_NTOT, _CH), _BF16),
                        pltpu.VMEM((_NTOT, _CH), _BF16),
                        pltpu.VMEM((_NTOT, _CH), _BF16),
                        pltpu.VMEM((_NTOT, _CH), _BF16),
                        pltpu.VMEM((_CH, _CH), _BF16),
                        pltpu.VMEM((_NQ, _NA + _NC), jnp.float32),
                        pltpu.VMEM((_NA, _NQ), jnp.float32),
                        pltpu.VMEM((_NC, _NQ), jnp.float32)],
        compiler_params=pltpu.CompilerParams(
            vmem_limit_bytes=60 * 1024 * 1024),
    )(*ins)
    return {"question": out[0], "answer": out[1], "concept": out[2]}
